# baseline (speedup 1.0000x reference)
import jax
import jax.numpy as jnp
from jax.experimental import pallas as pl
from jax.experimental.pallas import tpu as pltpu

_N0, _N1, _N2, _D, _G = 2048, 4096, 512, 32, 8
_BMA = 128
_S = _N1 // _BMA
_BMB1 = 512
_BM0 = 256
_S0 = _N0 // _BM0
_BMQ = 128
_BF = jnp.bfloat16


def _dot(a, b):
    return jax.lax.dot_general(a, b, (((1,), (0,)), ((), ())),
                               preferred_element_type=jnp.float32)


def _body(a1_ref, co_ref, b0_ref, a0_ref, b1_ref, e_ref, x_ref, s_ref,
          idx_ref, h0_ref, h1_ref, h2_ref, ge_ref,
          rhs_scr, b0h_scr, b1h_scr, xt_scr, w1t_scr,
          h0t_scr, h1t_scr, ut_scr, pt_scr, qt_scr):
    i = pl.program_id(0)

    @pl.when(i == 0)
    def _init():
        h0t_scr[...] = jnp.zeros_like(h0t_scr)
        h1t_scr[...] = jnp.zeros_like(h1t_scr)
        ut_scr[...] = jnp.zeros_like(ut_scr)
        pt_scr[...] = jnp.zeros_like(pt_scr)
        qt_scr[...] = jnp.zeros_like(qt_scr)
        xt_scr[...] = x_ref[...].T

    @pl.when(i < _S0)
    def _b1phase():
        b1h = b1_ref[...].astype(_BF)
        b1h_scr[pl.ds(i * _BMB1, _BMB1), :] = b1h
        t1b = _dot(b1h, (s_ref[...] * 0.5).astype(_BF))
        rhs_scr[pl.ds(i * _BMB1, _BMB1), :_D] = t1b.astype(_BF)
        rhs_scr[pl.ds(i * _BMB1, _BMB1), _D:] = (
            e_ref[pl.ds(i * _BMB1, _BMB1), :].astype(_BF))

    ebt = e_ref[pl.ds(i * _BMA, _BMA), :].T.astype(_BF)
    h1t_scr[...] += _dot(ebt, (a1_ref[...] + co_ref[...]).astype(_BF))

    @pl.when(jnp.logical_and(i >= _S0, i < 2 * _S0))
    def _phase1():
        j = i - _S0
        b0h = b0_ref[...].astype(_BF)
        b0h_scr[pl.ds(j * _BM0, _BM0), :] = b0h
        y = _dot(b0h, rhs_scr[...])
        yt = y.T
        t2bt = yt[:_D, :]
        xbt = xt_scr[:, pl.ds(j * _BM0, _BM0)]
        w1t_scr[:, pl.ds(j * _BM0, _BM0)] = t2bt + xbt
        lhs = jnp.concatenate([xbt, t2bt], axis=0).astype(_BF)
        z = _dot(lhs, a0_ref[...].astype(_BF))
        h0t_scr[...] += z[:_D, :]
        ut_scr[...] += z[_D:, :]
        h0t_scr[:, pl.ds(j * _BM0, _BM0)] += t2bt + yt[_D:, :]

    @pl.when(i >= 2 * _S0)
    def _phase2():
        j = i - 2 * _S0
        w1tb = w1t_scr[:, pl.ds(j * _BMQ, _BMQ)]
        xbt = xt_scr[:, pl.ds(j * _BMQ, _BMQ)]
        ubt = ut_scr[:, pl.ds(j * _BMQ, _BMQ)]
        w2tb = (ubt + xbt) * 0.5
        lhs = jnp.concatenate([w1tb, w2tb], axis=0).astype(_BF)
        pq = _dot(lhs, b0h_scr[pl.ds(j * _BMQ, _BMQ), :])
        pt_scr[...] += pq[:_D, :]
        qt_scr[...] += pq[_D:, :]

    @pl.when(i == _S - 1)
    def _fini():
        h1t = h1t_scr[...] + pt_scr[...]
        h1_ref[...] = h1t.T
        h0 = h0t_scr[...].T
        h0_ref[...] = h0
        h2t = _dot((qt_scr[...] + e_ref[...].T).astype(_BF), b1h_scr[...])
        h2_ref[...] = h2t.T
        idx = idx_ref[0, :]
        onehot = (jax.lax.broadcasted_iota(jnp.int32, (_G, _N0), 0)
                  == idx[None, :]).astype(jnp.float32)
        s = _dot(onehot, h0)
        cnt = jnp.sum(onehot, axis=1, keepdims=True)
        ge_ref[...] = s / jnp.maximum(cnt, 1.0)


def kernel(x, edge_attr, sse_attr, B0, B1, A0, A1, coA1, batch_idx):
    idx2d = batch_idx.reshape(1, _N0).astype(jnp.int32)
    h0, h1, h2, graph_emb = pl.pallas_call(
        _body,
        grid=(_S,),
        in_specs=[
            pl.BlockSpec((_BMA, _N1), lambda i: (i, 0)),
            pl.BlockSpec((_BMA, _N1), lambda i: (i, 0)),
            pl.BlockSpec((_BM0, _N1),
                         lambda i: (jnp.clip(i - _S0, 0, _S0 - 1), 0)),
            pl.BlockSpec((_BM0, _N0),
                         lambda i: (jnp.clip(i - _S0, 0, _S0 - 1), 0)),
            pl.BlockSpec((_BMB1, _N2), lambda i: (jnp.minimum(i, _S0 - 1), 0)),
            pl.BlockSpec((_N1, _D), lambda i: (0, 0)),
            pl.BlockSpec((_N0, _D), lambda i: (0, 0)),
            pl.BlockSpec((_N2, _D), lambda i: (0, 0)),
            pl.BlockSpec((1, _N0), lambda i: (0, 0)),
        ],
        out_specs=[
            pl.BlockSpec((_N0, _D), lambda i: (0, 0)),
            pl.BlockSpec((_N1, _D), lambda i: (0, 0)),
            pl.BlockSpec((_N2, _D), lambda i: (0, 0)),
            pl.BlockSpec((_G, _D), lambda i: (0, 0)),
        ],
        out_shape=[
            jax.ShapeDtypeStruct((_N0, _D), jnp.float32),
            jax.ShapeDtypeStruct((_N1, _D), jnp.float32),
            jax.ShapeDtypeStruct((_N2, _D), jnp.float32),
            jax.ShapeDtypeStruct((_G, _D), jnp.float32),
        ],
        scratch_shapes=[
            pltpu.VMEM((_N1, 2 * _D), _BF),
            pltpu.VMEM((_N0, _N1), _BF),
            pltpu.VMEM((_N1, _N2), _BF),
            pltpu.VMEM((_D, _N0), jnp.float32),
            pltpu.VMEM((_D, _N0), jnp.float32),
            pltpu.VMEM((_D, _N0), jnp.float32),
            pltpu.VMEM((_D, _N1), jnp.float32),
            pltpu.VMEM((_D, _N0), jnp.float32),
            pltpu.VMEM((_D, _N1), jnp.float32),
            pltpu.VMEM((_D, _N1), jnp.float32),
        ],
        compiler_params=pltpu.CompilerParams(dimension_semantics=("arbitrary",)),
    )(A1, coA1, B0, A0, B1, edge_attr, x, sse_attr, idx2d)
    return h0, h1, h2, graph_emb


# speedup vs baseline: 1.0095x; 1.0095x over previous
import jax
import jax.numpy as jnp
from jax.experimental import pallas as pl
from jax.experimental.pallas import tpu as pltpu

_N0, _N1, _N2, _D, _G = 2048, 4096, 512, 32, 8
_BMA = 128
_S = _N1 // _BMA
_BMB1 = 512
_BM0 = 256
_S0 = _N0 // _BM0
_BMQ = 256
_BF = jnp.bfloat16


def _dot(a, b):
    return jax.lax.dot_general(a, b, (((1,), (0,)), ((), ())),
                               preferred_element_type=jnp.float32)


def _body(a1_ref, co_ref, b0_ref, a0_ref, b1_ref, e_ref, x_ref, s_ref,
          idx_ref, h0_ref, h1_ref, h2_ref, ge_ref,
          rhs_scr, b0h_scr, b1h_scr, xt_scr, w1t_scr,
          h0t_scr, h1t_scr, ut_scr, pt_scr, qt_scr):
    i = pl.program_id(0)

    @pl.when(i == 0)
    def _init():
        h0t_scr[...] = jnp.zeros_like(h0t_scr)
        h1t_scr[...] = jnp.zeros_like(h1t_scr)
        ut_scr[...] = jnp.zeros_like(ut_scr)
        pt_scr[...] = jnp.zeros_like(pt_scr)
        qt_scr[...] = jnp.zeros_like(qt_scr)
        xt_scr[...] = x_ref[...].T

    @pl.when(i < _S0)
    def _b1phase():
        b1h = b1_ref[...].astype(_BF)
        b1h_scr[pl.ds(i * _BMB1, _BMB1), :] = b1h
        t1b = _dot(b1h, (s_ref[...] * 0.5).astype(_BF))
        rhs_scr[pl.ds(i * _BMB1, _BMB1), :_D] = t1b.astype(_BF)
        rhs_scr[pl.ds(i * _BMB1, _BMB1), _D:] = (
            e_ref[pl.ds(i * _BMB1, _BMB1), :].astype(_BF))

    ebt = e_ref[pl.ds(i * _BMA, _BMA), :].T.astype(_BF)
    h1t_scr[...] += _dot(ebt, (a1_ref[...] + co_ref[...]).astype(_BF))

    @pl.when(jnp.logical_and(i >= _S0, i < 2 * _S0))
    def _phase1():
        j = i - _S0
        b0h = b0_ref[...].astype(_BF)
        b0h_scr[pl.ds(j * _BM0, _BM0), :] = b0h
        y = _dot(b0h, rhs_scr[...])
        yt = y.T
        t2bt = yt[:_D, :]
        xbt = xt_scr[:, pl.ds(j * _BM0, _BM0)]
        w1t_scr[:, pl.ds(j * _BM0, _BM0)] = t2bt + xbt
        lhs = jnp.concatenate([xbt, t2bt], axis=0).astype(_BF)
        z = _dot(lhs, a0_ref[...].astype(_BF))
        h0t_scr[...] += z[:_D, :]
        ut_scr[...] += z[_D:, :]
        h0t_scr[:, pl.ds(j * _BM0, _BM0)] += t2bt + yt[_D:, :]

    @pl.when(jnp.logical_and(i >= 2 * _S0, i < 3 * _S0))
    def _phase2():
        j = i - 2 * _S0
        w1tb = w1t_scr[:, pl.ds(j * _BMQ, _BMQ)]
        xbt = xt_scr[:, pl.ds(j * _BMQ, _BMQ)]
        ubt = ut_scr[:, pl.ds(j * _BMQ, _BMQ)]
        w2tb = (ubt + xbt) * 0.5
        lhs = jnp.concatenate([w1tb, w2tb], axis=0).astype(_BF)
        pq = _dot(lhs, b0h_scr[pl.ds(j * _BMQ, _BMQ), :])
        pt_scr[...] += pq[:_D, :]
        qt_scr[...] += pq[_D:, :]

    @pl.when(i == _S - 1)
    def _fini():
        h1t = h1t_scr[...] + pt_scr[...]
        h1_ref[...] = h1t.T
        h0 = h0t_scr[...].T
        h0_ref[...] = h0
        h2t = _dot((qt_scr[...] + e_ref[...].T).astype(_BF), b1h_scr[...])
        h2_ref[...] = h2t.T
        idx = idx_ref[0, :]
        onehot = (jax.lax.broadcasted_iota(jnp.int32, (_G, _N0), 0)
                  == idx[None, :]).astype(jnp.float32)
        s = _dot(onehot, h0)
        cnt = jnp.sum(onehot, axis=1, keepdims=True)
        ge_ref[...] = s / jnp.maximum(cnt, 1.0)


def kernel(x, edge_attr, sse_attr, B0, B1, A0, A1, coA1, batch_idx):
    idx2d = batch_idx.reshape(1, _N0).astype(jnp.int32)
    h0, h1, h2, graph_emb = pl.pallas_call(
        _body,
        grid=(_S,),
        in_specs=[
            pl.BlockSpec((_BMA, _N1), lambda i: (i, 0)),
            pl.BlockSpec((_BMA, _N1), lambda i: (i, 0)),
            pl.BlockSpec((_BM0, _N1),
                         lambda i: (jnp.clip(i - _S0, 0, _S0 - 1), 0)),
            pl.BlockSpec((_BM0, _N0),
                         lambda i: (jnp.clip(i - _S0, 0, _S0 - 1), 0)),
            pl.BlockSpec((_BMB1, _N2), lambda i: (jnp.minimum(i, _S0 - 1), 0)),
            pl.BlockSpec((_N1, _D), lambda i: (0, 0)),
            pl.BlockSpec((_N0, _D), lambda i: (0, 0)),
            pl.BlockSpec((_N2, _D), lambda i: (0, 0)),
            pl.BlockSpec((1, _N0), lambda i: (0, 0)),
        ],
        out_specs=[
            pl.BlockSpec((_N0, _D), lambda i: (0, 0)),
            pl.BlockSpec((_N1, _D), lambda i: (0, 0)),
            pl.BlockSpec((_N2, _D), lambda i: (0, 0)),
            pl.BlockSpec((_G, _D), lambda i: (0, 0)),
        ],
        out_shape=[
            jax.ShapeDtypeStruct((_N0, _D), jnp.float32),
            jax.ShapeDtypeStruct((_N1, _D), jnp.float32),
            jax.ShapeDtypeStruct((_N2, _D), jnp.float32),
            jax.ShapeDtypeStruct((_G, _D), jnp.float32),
        ],
        scratch_shapes=[
            pltpu.VMEM((_N1, 2 * _D), _BF),
            pltpu.VMEM((_N0, _N1), _BF),
            pltpu.VMEM((_N1, _N2), _BF),
            pltpu.VMEM((_D, _N0), jnp.float32),
            pltpu.VMEM((_D, _N0), jnp.float32),
            pltpu.VMEM((_D, _N0), jnp.float32),
            pltpu.VMEM((_D, _N1), jnp.float32),
            pltpu.VMEM((_D, _N0), jnp.float32),
            pltpu.VMEM((_D, _N1), jnp.float32),
            pltpu.VMEM((_D, _N1), jnp.float32),
        ],
        compiler_params=pltpu.CompilerParams(dimension_semantics=("arbitrary",)),
    )(A1, coA1, B0, A0, B1, edge_attr, x, sse_attr, idx2d)
    return h0, h1, h2, graph_emb


# speedup vs baseline: 1.0110x; 1.0015x over previous
import jax
import jax.numpy as jnp
from jax.experimental import pallas as pl
from jax.experimental.pallas import tpu as pltpu

_N0, _N1, _N2, _D, _G = 2048, 4096, 512, 32, 8
_BMA = 128
_S = _N1 // _BMA
_BMB1 = 512
_BM0 = 256
_S0 = _N0 // _BM0
_BMQ = 256
_BF = jnp.bfloat16


def _dot(a, b):
    return jax.lax.dot_general(a, b, (((1,), (0,)), ((), ())),
                               preferred_element_type=jnp.float32)


def _body(a1_ref, co_ref, b0_ref, a0_ref, b1_ref, e_ref, x_ref, s_ref,
          idx_ref, h0_ref, h1_ref, h2_ref, ge_ref,
          rhs_scr, b0h_scr, b1h_scr, xt_scr, et_scr, w1t_scr,
          h0t_scr, h1t_scr, ut_scr, pt_scr, qt_scr):
    i = pl.program_id(0)

    @pl.when(i == 0)
    def _init():
        h0t_scr[...] = jnp.zeros_like(h0t_scr)
        h1t_scr[...] = jnp.zeros_like(h1t_scr)
        ut_scr[...] = jnp.zeros_like(ut_scr)
        pt_scr[...] = jnp.zeros_like(pt_scr)
        qt_scr[...] = jnp.zeros_like(qt_scr)
        xt_scr[...] = x_ref[...].T
        et_scr[...] = e_ref[...].T.astype(_BF)

    @pl.when(i < _S0)
    def _b1phase():
        b1h = b1_ref[...].astype(_BF)
        b1h_scr[pl.ds(i * _BMB1, _BMB1), :] = b1h
        t1b = _dot(b1h, (s_ref[...] * 0.5).astype(_BF))
        rhs_scr[pl.ds(i * _BMB1, _BMB1), :_D] = t1b.astype(_BF)
        rhs_scr[pl.ds(i * _BMB1, _BMB1), _D:] = (
            e_ref[pl.ds(i * _BMB1, _BMB1), :].astype(_BF))

    ebt = et_scr[:, pl.ds(i * _BMA, _BMA)]
    h1t_scr[...] += _dot(ebt, (a1_ref[...] + co_ref[...]).astype(_BF))

    @pl.when(jnp.logical_and(i >= _S0, i < 2 * _S0))
    def _phase1():
        j = i - _S0
        b0h = b0_ref[...].astype(_BF)
        b0h_scr[pl.ds(j * _BM0, _BM0), :] = b0h
        y = _dot(b0h, rhs_scr[...])
        yt = y.T
        t2bt = yt[:_D, :]
        xbt = xt_scr[:, pl.ds(j * _BM0, _BM0)]
        w1t_scr[:, pl.ds(j * _BM0, _BM0)] = t2bt + xbt
        lhs = jnp.concatenate([xbt, t2bt], axis=0).astype(_BF)
        z = _dot(lhs, a0_ref[...].astype(_BF))
        h0t_scr[...] += z[:_D, :]
        ut_scr[...] += z[_D:, :]
        h0t_scr[:, pl.ds(j * _BM0, _BM0)] += t2bt + yt[_D:, :]

    @pl.when(jnp.logical_and(i >= 2 * _S0, i < 3 * _S0))
    def _phase2():
        j = i - 2 * _S0
        w1tb = w1t_scr[:, pl.ds(j * _BMQ, _BMQ)]
        xbt = xt_scr[:, pl.ds(j * _BMQ, _BMQ)]
        ubt = ut_scr[:, pl.ds(j * _BMQ, _BMQ)]
        w2tb = (ubt + xbt) * 0.5
        lhs = jnp.concatenate([w1tb, w2tb], axis=0).astype(_BF)
        pq = _dot(lhs, b0h_scr[pl.ds(j * _BMQ, _BMQ), :])
        pt_scr[...] += pq[:_D, :]
        qt_scr[...] += pq[_D:, :]

    @pl.when(i == _S - 1)
    def _fini():
        h1t = h1t_scr[...] + pt_scr[...]
        h1_ref[...] = h1t.T
        h0 = h0t_scr[...].T
        h0_ref[...] = h0
        h2t = _dot((qt_scr[...] + e_ref[...].T).astype(_BF), b1h_scr[...])
        h2_ref[...] = h2t.T
        idx = idx_ref[0, :]
        onehot = (jax.lax.broadcasted_iota(jnp.int32, (_G, _N0), 0)
                  == idx[None, :]).astype(jnp.float32)
        s = _dot(onehot, h0)
        cnt = jnp.sum(onehot, axis=1, keepdims=True)
        ge_ref[...] = s / jnp.maximum(cnt, 1.0)


def kernel(x, edge_attr, sse_attr, B0, B1, A0, A1, coA1, batch_idx):
    idx2d = batch_idx.reshape(1, _N0).astype(jnp.int32)
    h0, h1, h2, graph_emb = pl.pallas_call(
        _body,
        grid=(_S,),
        in_specs=[
            pl.BlockSpec((_BMA, _N1), lambda i: (i, 0)),
            pl.BlockSpec((_BMA, _N1), lambda i: (i, 0)),
            pl.BlockSpec((_BM0, _N1),
                         lambda i: (jnp.clip(i - _S0, 0, _S0 - 1), 0)),
            pl.BlockSpec((_BM0, _N0),
                         lambda i: (jnp.clip(i - _S0, 0, _S0 - 1), 0)),
            pl.BlockSpec((_BMB1, _N2), lambda i: (jnp.minimum(i, _S0 - 1), 0)),
            pl.BlockSpec((_N1, _D), lambda i: (0, 0)),
            pl.BlockSpec((_N0, _D), lambda i: (0, 0)),
            pl.BlockSpec((_N2, _D), lambda i: (0, 0)),
            pl.BlockSpec((1, _N0), lambda i: (0, 0)),
        ],
        out_specs=[
            pl.BlockSpec((_N0, _D), lambda i: (0, 0)),
            pl.BlockSpec((_N1, _D), lambda i: (0, 0)),
            pl.BlockSpec((_N2, _D), lambda i: (0, 0)),
            pl.BlockSpec((_G, _D), lambda i: (0, 0)),
        ],
        out_shape=[
            jax.ShapeDtypeStruct((_N0, _D), jnp.float32),
            jax.ShapeDtypeStruct((_N1, _D), jnp.float32),
            jax.ShapeDtypeStruct((_N2, _D), jnp.float32),
            jax.ShapeDtypeStruct((_G, _D), jnp.float32),
        ],
        scratch_shapes=[
            pltpu.VMEM((_N1, 2 * _D), _BF),
            pltpu.VMEM((_N0, _N1), _BF),
            pltpu.VMEM((_N1, _N2), _BF),
            pltpu.VMEM((_D, _N0), jnp.float32),
            pltpu.VMEM((_D, _N1), _BF),
            pltpu.VMEM((_D, _N0), jnp.float32),
            pltpu.VMEM((_D, _N0), jnp.float32),
            pltpu.VMEM((_D, _N1), jnp.float32),
            pltpu.VMEM((_D, _N0), jnp.float32),
            pltpu.VMEM((_D, _N1), jnp.float32),
            pltpu.VMEM((_D, _N1), jnp.float32),
        ],
        compiler_params=pltpu.CompilerParams(dimension_semantics=("arbitrary",)),
    )(A1, coA1, B0, A0, B1, edge_attr, x, sse_attr, idx2d)
    return h0, h1, h2, graph_emb


# speedup vs baseline: 1.0146x; 1.0036x over previous
import jax
import jax.numpy as jnp
from jax.experimental import pallas as pl
from jax.experimental.pallas import tpu as pltpu

_N0, _N1, _N2, _D, _G = 2048, 4096, 512, 32, 8
_BMA = 128
_S = _N1 // _BMA
_BMB1 = 512
_BM0 = 256
_S0 = _N0 // _BM0
_BMQ = 256
_BF = jnp.bfloat16


def _dot(a, b):
    return jax.lax.dot_general(a, b, (((1,), (0,)), ((), ())),
                               preferred_element_type=jnp.float32)


def _body(a1_ref, co_ref, b0_ref, a0_ref, b1_ref, e_ref, x_ref, s_ref,
          idx_ref, h0_ref, h1_ref, h2_ref, ge_ref,
          rhs_scr, b0h_scr, b1h_scr, xt_scr, w1t_scr,
          h0t_scr, h1t_scr, ut_scr, pt_scr, qt_scr):
    i = pl.program_id(0)

    @pl.when(i == 0)
    def _init():
        h0t_scr[...] = jnp.zeros_like(h0t_scr)
        h1t_scr[...] = jnp.zeros_like(h1t_scr)
        ut_scr[...] = jnp.zeros_like(ut_scr)
        pt_scr[...] = jnp.zeros_like(pt_scr)
        qt_scr[...] = jnp.zeros_like(qt_scr)
        xt_scr[...] = x_ref[...].T

    @pl.when(i < _S0)
    def _b1phase():
        b1h = b1_ref[...].astype(_BF)
        b1h_scr[pl.ds(i * _BMB1, _BMB1), :] = b1h
        t1b = _dot(b1h, (s_ref[...] * 0.5).astype(_BF))
        rhs_scr[pl.ds(i * _BMB1, _BMB1), :_D] = t1b.astype(_BF)
        rhs_scr[pl.ds(i * _BMB1, _BMB1), _D:] = (
            e_ref[pl.ds(i * _BMB1, _BMB1), :].astype(_BF))

    ebt = e_ref[pl.ds(i * _BMA, _BMA), :].T.astype(_BF)
    h1t_scr[...] += _dot(ebt, (a1_ref[...] + co_ref[...]).astype(_BF))

    @pl.when(jnp.logical_and(i >= _S0, i < 2 * _S0))
    def _phase1():
        j = i - _S0
        b0h = b0_ref[...].astype(_BF)
        b0h_scr[pl.ds(j * _BM0, _BM0), :] = b0h
        y = _dot(b0h, rhs_scr[...])
        yt = y.T
        t2bt = yt[:_D, :]
        xbt = xt_scr[:, pl.ds(j * _BM0, _BM0)]
        w1t_scr[:, pl.ds(j * _BM0, _BM0)] = t2bt + xbt
        lhs = jnp.concatenate([xbt, t2bt], axis=0).astype(_BF)
        z = _dot(lhs, a0_ref[...].astype(_BF))
        h0t_scr[...] += z[:_D, :]
        ut_scr[...] += z[_D:, :]
        h0t_scr[:, pl.ds(j * _BM0, _BM0)] += t2bt + yt[_D:, :]

    @pl.when(jnp.logical_and(i >= 2 * _S0, i < 3 * _S0))
    def _phase2():
        j = i - 2 * _S0
        w1tb = w1t_scr[:, pl.ds(j * _BMQ, _BMQ)]
        xbt = xt_scr[:, pl.ds(j * _BMQ, _BMQ)]
        ubt = ut_scr[:, pl.ds(j * _BMQ, _BMQ)]
        w2tb = (ubt + xbt) * 0.5
        lhs = jnp.concatenate([w1tb, w2tb], axis=0).astype(_BF)
        pq = _dot(lhs, b0h_scr[pl.ds(j * _BMQ, _BMQ), :])
        pt_scr[...] += pq[:_D, :]
        qt_scr[...] += pq[_D:, :]

    @pl.when(i == _S - 1)
    def _fini():
        h1t = h1t_scr[...] + pt_scr[...]
        h1_ref[...] = h1t.T
        h0 = h0t_scr[...].T
        h0_ref[...] = h0
        h2t = _dot((qt_scr[...] + e_ref[...].T).astype(_BF), b1h_scr[...])
        h2_ref[...] = h2t.T
        idx = idx_ref[0, :]
        onehot = (jax.lax.broadcasted_iota(jnp.int32, (_G, _N0), 0)
                  == idx[None, :]).astype(jnp.float32)
        s = _dot(onehot, h0)
        cnt = jnp.sum(onehot, axis=1, keepdims=True)
        ge_ref[...] = s / jnp.maximum(cnt, 1.0)


def kernel(x, edge_attr, sse_attr, B0, B1, A0, A1, coA1, batch_idx):
    idx2d = batch_idx.reshape(1, _N0).astype(jnp.int32)
    h0, h1, h2, graph_emb = pl.pallas_call(
        _body,
        grid=(_S,),
        in_specs=[
            pl.BlockSpec((_BMA, _N1), lambda i: (i, 0)),
            pl.BlockSpec((_BMA, _N1), lambda i: (i, 0)),
            pl.BlockSpec((_BM0, _N1),
                         lambda i: (jnp.clip(i - _S0, 0, _S0 - 1), 0)),
            pl.BlockSpec((_BM0, _N0),
                         lambda i: (jnp.clip(i - _S0, 0, _S0 - 1), 0)),
            pl.BlockSpec((_BMB1, _N2), lambda i: (jnp.minimum(i, _S0 - 1), 0)),
            pl.BlockSpec((_N1, _D), lambda i: (0, 0)),
            pl.BlockSpec((_N0, _D), lambda i: (0, 0)),
            pl.BlockSpec((_N2, _D), lambda i: (0, 0)),
            pl.BlockSpec((1, _N0), lambda i: (0, 0)),
        ],
        out_specs=[
            pl.BlockSpec((_N0, _D), lambda i: (0, 0)),
            pl.BlockSpec((_N1, _D), lambda i: (0, 0)),
            pl.BlockSpec((_N2, _D), lambda i: (0, 0)),
            pl.BlockSpec((_G, _D), lambda i: (0, 0)),
        ],
        out_shape=[
            jax.ShapeDtypeStruct((_N0, _D), jnp.float32),
            jax.ShapeDtypeStruct((_N1, _D), jnp.float32),
            jax.ShapeDtypeStruct((_N2, _D), jnp.float32),
            jax.ShapeDtypeStruct((_G, _D), jnp.float32),
        ],
        scratch_shapes=[
            pltpu.VMEM((_N1, 2 * _D), _BF),
            pltpu.VMEM((_N0, _N1), _BF),
            pltpu.VMEM((_N1, _N2), _BF),
            pltpu.VMEM((_D, _N0), jnp.float32),
            pltpu.VMEM((_D, _N0), jnp.float32),
            pltpu.VMEM((_D, _N0), jnp.float32),
            pltpu.VMEM((_D, _N1), jnp.float32),
            pltpu.VMEM((_D, _N0), jnp.float32),
            pltpu.VMEM((_D, _N1), jnp.float32),
            pltpu.VMEM((_D, _N1), jnp.float32),
        ],
        compiler_params=pltpu.CompilerParams(dimension_semantics=("arbitrary",)),
    )(A1, coA1, B0, A0, B1, edge_attr, x, sse_attr, idx2d)
    return h0, h1, h2, graph_emb
